# baseline (speedup 1.0000x reference)
import functools

import jax
import jax.numpy as jnp
from jax import lax
from jax.experimental import pallas as pl
from jax.experimental.pallas import tpu as pltpu
from jax.experimental.pallas import tpu_sc as plsc

N_TRI = 1048576
N_ENT = 1000000
N_REL = 1000

NC = 2
NS = 16
NW = NC * NS
CHUNK = 8192

ENT_STRIPE = 62464
ENT_BULK = NS * ENT_STRIPE
ENT2_PAD = 640
TAB_PAD = 1000448
REL_PAD = 1024

R_UNROLL = 4


def _make_sc_gather(n_rows):
    mesh = plsc.VectorSubcoreMesh(core_axis_name="c", subcore_axis_name="s")
    col = jax.ShapeDtypeStruct((n_rows,), jnp.int32)
    PER_W = n_rows // NW
    STEPS = PER_W // CHUNK
    NCHUNK = 2 * STEPS
    HALF_R = PER_W // 2
    R_SLICE = HALF_R // STEPS

    @functools.partial(
        pl.kernel,
        mesh=mesh,
        compiler_params=pltpu.CompilerParams(needs_layout_passes=False),
        out_type=(col, col, col),
        scratch_types=[
            pltpu.VMEM((CHUNK,), jnp.int32),
            pltpu.VMEM((CHUNK,), jnp.int32),
            pltpu.VMEM((CHUNK,), jnp.int32),
            pltpu.VMEM((CHUNK,), jnp.int32),
            pltpu.VMEM((HALF_R,), jnp.int32),
            pltpu.VMEM((HALF_R,), jnp.int32),
            pltpu.VMEM((REL_PAD,), jnp.int32),
            pltpu.VMEM_SHARED((TAB_PAD,), jnp.int32),
            pltpu.SemaphoreType.DMA,
            pltpu.SemaphoreType.DMA,
            pltpu.SemaphoreType.DMA,
            pltpu.SemaphoreType.DMA,
            pltpu.SemaphoreType.DMA,
            pltpu.SemaphoreType.DMA,
            pltpu.SemaphoreType.DMA,
            pltpu.SemaphoreType.DMA,
        ],
    )
    def body(h_hbm, r_hbm, t_hbm, ent_hbm, ent2_hbm, rel_hbm,
             ho_hbm, ro_hbm, to_hbm,
             idx0, idx1, val0, val1, ridx, rval, rel_v, tab_sh,
             si0, si1, sg0, sg1, so0, so1, srin, srout):
        cid = lax.axis_index("c")
        sid = lax.axis_index("s")
        wid = sid * NC + cid
        base = wid * PER_W

        idx = (idx0, idx1)
        val = (val0, val1)
        s_in = (si0, si1)
        s_g = (sg0, sg1)
        s_out = (so0, so1)

        srcs = (h_hbm, t_hbm)
        dsts = (ho_hbm, to_hbm)

        def src_slice(k):
            s, i = divmod(k, STEPS)
            return srcs[s].at[pl.ds(base + i * CHUNK, CHUNK)]

        def dst_slice(k):
            s, i = divmod(k, STEPS)
            return dsts[s].at[pl.ds(base + i * CHUNK, CHUNK)]

        ins = [None] * (NCHUNK + 2)
        outs = [None] * NCHUNK
        gs = [None] * NCHUNK

        ins[0] = pltpu.async_copy(src_slice(0), idx[0], s_in[0])
        ins[1] = pltpu.async_copy(src_slice(1), idx[1], s_in[1])
        r_in = pltpu.async_copy(r_hbm.at[pl.ds(base, HALF_R)], ridx, srin)

        so = sid * ENT_STRIPE
        pltpu.sync_copy(ent_hbm.at[pl.ds(so, ENT_STRIPE)],
                        tab_sh.at[pl.ds(so, ENT_STRIPE)])

        @pl.when(sid == 1)
        def _():
            pltpu.sync_copy(ent2_hbm, tab_sh.at[pl.ds(ENT_BULK, ENT2_PAD)])

        pltpu.sync_copy(rel_hbm, rel_v)
        plsc.subcore_barrier()

        ins[0].wait()
        gs[0] = pltpu.async_copy(tab_sh.at[idx[0]], val[0], s_g[0])
        r_in.wait()

        def r_slice(k):
            r0 = (k % STEPS) * R_SLICE

            def it(j, _):
                for u in range(R_UNROLL):
                    s = r0 + (j * R_UNROLL + u) * 16
                    v = ridx[pl.ds(s, 16)]
                    rval[pl.ds(s, 16)] = plsc.load_gather(rel_v, [v])
                return 0

            lax.fori_loop(0, R_SLICE // (16 * R_UNROLL), it, 0)

        r_out0 = None
        for k in range(NCHUNK):
            b = k % 2
            nb = (k + 1) % 2
            if k + 1 < NCHUNK:
                ins[k + 1].wait()
                if k >= 1:
                    outs[k - 1].wait()
                gs[k + 1] = pltpu.async_copy(
                    tab_sh.at[idx[nb]], val[nb], s_g[nb])
            if k == STEPS:
                r_in.wait()
                r_out0.wait()
            r_slice(k)
            if k == STEPS - 1:
                r_out0 = pltpu.async_copy(
                    rval, ro_hbm.at[pl.ds(base, HALF_R)], srout)
                r_in = pltpu.async_copy(
                    r_hbm.at[pl.ds(base + HALF_R, HALF_R)], ridx, srin)
            gs[k].wait()
            outs[k] = pltpu.async_copy(val[b], dst_slice(k), s_out[b])
            if k + 2 < NCHUNK:
                ins[k + 2] = pltpu.async_copy(src_slice(k + 2), idx[b], s_in[b])

        r_out1 = pltpu.async_copy(
            rval, ro_hbm.at[pl.ds(base + HALF_R, HALF_R)], srout)
        outs[NCHUNK - 2].wait()
        outs[NCHUNK - 1].wait()
        r_out1.wait()

    return body


HALF = N_TRI // 2
_sc_gather_half = _make_sc_gather(HALF)


def kernel(raw_triples, entity_map, relation_map):
    raw_triples = raw_triples.astype(jnp.int32)
    ent = entity_map.astype(jnp.int32)
    ent2 = jnp.pad(ent[ENT_BULK:], (0, ENT2_PAD - (N_ENT - ENT_BULK)))
    rel = jnp.pad(relation_map.astype(jnp.int32), (0, REL_PAD - N_REL))

    rawA = raw_triples[:HALF]
    encA = _sc_gather_half(rawA[:, 0], rawA[:, 1], rawA[:, 2],
                           ent, ent2, rel)

    rawB = lax.optimization_barrier(raw_triples)[HALF:]
    encB = _sc_gather_half(rawB[:, 0], rawB[:, 1], rawB[:, 2],
                           ent, ent2, rel)

    outA = jnp.stack(encA, axis=1)
    outB = jnp.stack(encB, axis=1)
    return jnp.concatenate((outA, outB), axis=0)


# speedup vs baseline: 1.2526x; 1.2526x over previous
import functools

import jax
import jax.numpy as jnp
from jax import lax
from jax.experimental import pallas as pl
from jax.experimental.pallas import tpu as pltpu
from jax.experimental.pallas import tpu_sc as plsc

N_TRI = 1048576
N_ENT = 1000000
N_REL = 1000

NC = 2
NS = 16
NW = NC * NS
PER_W = N_TRI // NW
CHUNK = 4096
NBUF = 3
STEPS = PER_W // CHUNK
NCHUNK = 2 * STEPS

ENT_STRIPE = 62464
ENT_BULK = NS * ENT_STRIPE
ENT2_PAD = 640
TAB_PAD = 1000448
REL_PAD = 1024

R_UNROLL = 4
HALF_R = PER_W // 2
R_SLICE = PER_W // NCHUNK


def _make_sc_gather():
    mesh = plsc.VectorSubcoreMesh(core_axis_name="c", subcore_axis_name="s")
    col = jax.ShapeDtypeStruct((N_TRI,), jnp.int32)

    @functools.partial(
        pl.kernel,
        mesh=mesh,
        compiler_params=pltpu.CompilerParams(needs_layout_passes=False),
        out_type=(col, col, col),
        scratch_types=(
            [pltpu.VMEM((CHUNK,), jnp.int32)] * (2 * NBUF)
            + [
                pltpu.VMEM((HALF_R,), jnp.int32),
                pltpu.VMEM((HALF_R,), jnp.int32),
                pltpu.VMEM((REL_PAD,), jnp.int32),
                pltpu.VMEM_SHARED((TAB_PAD,), jnp.int32),
            ]
            + [pltpu.SemaphoreType.DMA] * (3 * NBUF + 2)
        ),
    )
    def body(h_hbm, r_hbm, t_hbm, ent_hbm, ent2_hbm, rel_hbm,
             ho_hbm, ro_hbm, to_hbm, *scratch):
        idx = scratch[0:NBUF]
        val = scratch[NBUF:2 * NBUF]
        ridx, rval, rel_v, tab_sh = scratch[2 * NBUF:2 * NBUF + 4]
        sems = scratch[2 * NBUF + 4:]
        s_in = sems[0:NBUF]
        s_g = sems[NBUF:2 * NBUF]
        s_out = sems[2 * NBUF:3 * NBUF]
        srin, srout = sems[3 * NBUF:]
        cid = lax.axis_index("c")
        sid = lax.axis_index("s")
        wid = sid * NC + cid
        base = wid * PER_W

        srcs = (h_hbm, t_hbm)
        dsts = (ho_hbm, to_hbm)

        def src_slice(k):
            s, i = divmod(k, STEPS)
            return srcs[s].at[pl.ds(base + i * CHUNK, CHUNK)]

        def dst_slice(k):
            s, i = divmod(k, STEPS)
            return dsts[s].at[pl.ds(base + i * CHUNK, CHUNK)]

        ins = [None] * (NCHUNK + NBUF)
        outs = [None] * NCHUNK
        gs = [None] * NCHUNK

        for k in range(NBUF):
            ins[k] = pltpu.async_copy(src_slice(k), idx[k], s_in[k])
        r_in = pltpu.async_copy(r_hbm.at[pl.ds(base, HALF_R)], ridx, srin)

        so = sid * ENT_STRIPE
        pltpu.sync_copy(ent_hbm.at[pl.ds(so, ENT_STRIPE)],
                        tab_sh.at[pl.ds(so, ENT_STRIPE)])

        @pl.when(sid == 1)
        def _():
            pltpu.sync_copy(ent2_hbm, tab_sh.at[pl.ds(ENT_BULK, ENT2_PAD)])

        pltpu.sync_copy(rel_hbm, rel_v)
        plsc.subcore_barrier()

        r_in.wait()

        def r_slice(k):
            r0 = (k % (NCHUNK // 2)) * R_SLICE

            def it(j, _):
                for u in range(R_UNROLL):
                    s = r0 + (j * R_UNROLL + u) * 16
                    v = ridx[pl.ds(s, 16)]
                    rval[pl.ds(s, 16)] = plsc.load_gather(rel_v, [v])
                return 0

            lax.fori_loop(0, R_SLICE // (16 * R_UNROLL), it, 0)

        r_out0 = None
        HALF_STEP = NCHUNK // 2
        for k in range(NCHUNK):
            b = k % NBUF
            ins[k].wait()
            if k >= NBUF:
                outs[k - NBUF].wait()
            gs[k] = pltpu.async_copy(tab_sh.at[idx[b]], val[b], s_g[b])
            if k == HALF_STEP:
                r_in.wait()
                r_out0.wait()
            r_slice(k)
            if k == HALF_STEP - 1:
                r_out0 = pltpu.async_copy(
                    rval, ro_hbm.at[pl.ds(base, HALF_R)], srout)
                r_in = pltpu.async_copy(
                    r_hbm.at[pl.ds(base + HALF_R, HALF_R)], ridx, srin)
            if k >= NBUF - 1:
                j = k - (NBUF - 1)
                gs[j].wait()
                jb = j % NBUF
                outs[j] = pltpu.async_copy(val[jb], dst_slice(j), s_out[jb])
                if k + 1 < NCHUNK:
                    ins[k + 1] = pltpu.async_copy(
                        src_slice(k + 1), idx[jb], s_in[jb])

        for j in range(NCHUNK - (NBUF - 1), NCHUNK):
            gs[j].wait()
            jb = j % NBUF
            outs[j] = pltpu.async_copy(val[jb], dst_slice(j), s_out[jb])

        r_out1 = pltpu.async_copy(
            rval, ro_hbm.at[pl.ds(base + HALF_R, HALF_R)], srout)
        for j in range(NCHUNK - NBUF, NCHUNK):
            outs[j].wait()
        r_out1.wait()

    return body


_sc_gather = _make_sc_gather()


def kernel(raw_triples, entity_map, relation_map):
    raw_triples = raw_triples.astype(jnp.int32)
    h = raw_triples[:, 0]
    r = raw_triples[:, 1]
    t = raw_triples[:, 2]
    ent = entity_map.astype(jnp.int32)
    ent2 = jnp.pad(ent[ENT_BULK:], (0, ENT2_PAD - (N_ENT - ENT_BULK)))
    rel = jnp.pad(relation_map.astype(jnp.int32), (0, REL_PAD - N_REL))
    h_enc, r_enc, t_enc = _sc_gather(h, r, t, ent, ent2, rel)
    return jnp.stack((h_enc, r_enc, t_enc), axis=1)


# speedup vs baseline: 1.2753x; 1.0181x over previous
import functools

import jax
import jax.numpy as jnp
from jax import lax
from jax.experimental import pallas as pl
from jax.experimental.pallas import tpu as pltpu
from jax.experimental.pallas import tpu_sc as plsc

N_TRI = 1048576
N_ENT = 1000000
N_REL = 1000

NC = 2
NS = 16
NW = NC * NS
PER_W = N_TRI // NW
CHUNK = 8192
STEPS = PER_W // CHUNK
NCHUNK = 2 * STEPS

ENT_STRIPE = 62464
ENT_BULK = NS * ENT_STRIPE
ENT2_PAD = 640
TAB_PAD = 1000448
REL_PAD = 1024

R_UNROLL = 4
HALF_R = PER_W // 2
R_SLICE = HALF_R // STEPS


def _make_sc_gather():
    mesh = plsc.VectorSubcoreMesh(core_axis_name="c", subcore_axis_name="s")
    col = jax.ShapeDtypeStruct((N_TRI,), jnp.int32)

    @functools.partial(
        pl.kernel,
        mesh=mesh,
        compiler_params=pltpu.CompilerParams(needs_layout_passes=False),
        out_type=(col, col, col),
        scratch_types=[
            pltpu.VMEM((CHUNK,), jnp.int32),
            pltpu.VMEM((CHUNK,), jnp.int32),
            pltpu.VMEM((CHUNK,), jnp.int32),
            pltpu.VMEM((CHUNK,), jnp.int32),
            pltpu.VMEM((HALF_R,), jnp.int32),
            pltpu.VMEM((HALF_R,), jnp.int32),
            pltpu.VMEM((REL_PAD,), jnp.int32),
            pltpu.VMEM_SHARED((TAB_PAD,), jnp.int32),
            pltpu.SemaphoreType.DMA,
            pltpu.SemaphoreType.DMA,
            pltpu.SemaphoreType.DMA,
            pltpu.SemaphoreType.DMA,
            pltpu.SemaphoreType.DMA,
            pltpu.SemaphoreType.DMA,
            pltpu.SemaphoreType.DMA,
            pltpu.SemaphoreType.DMA,
        ],
    )
    def body(h_hbm, r_hbm, t_hbm, ent_hbm, ent2_hbm, rel_hbm,
             ho_hbm, ro_hbm, to_hbm,
             idx0, idx1, val0, val1, ridx, rval, rel_v, tab_sh,
             si0, si1, sg0, sg1, so0, so1, srin, srout):
        cid = lax.axis_index("c")
        sid = lax.axis_index("s")
        wid = sid * NC + cid
        base = wid * PER_W

        idx = (idx0, idx1)
        val = (val0, val1)
        s_in = (si0, si1)
        s_g = (sg0, sg1)
        s_out = (so0, so1)

        srcs = (h_hbm, t_hbm)
        dsts = (ho_hbm, to_hbm)

        def src_slice(k):
            s, i = divmod(k, STEPS)
            return srcs[s].at[pl.ds(base + i * CHUNK, CHUNK)]

        def dst_slice(k):
            s, i = divmod(k, STEPS)
            return dsts[s].at[pl.ds(base + i * CHUNK, CHUNK)]

        ins = [None] * (NCHUNK + 2)
        outs = [None] * NCHUNK
        gs = [None] * NCHUNK

        ins[0] = pltpu.async_copy(src_slice(0), idx[0], s_in[0])
        ins[1] = pltpu.async_copy(src_slice(1), idx[1], s_in[1])
        r_in = pltpu.async_copy(r_hbm.at[pl.ds(base, HALF_R)], ridx, srin)

        so = sid * ENT_STRIPE
        pltpu.sync_copy(ent_hbm.at[pl.ds(so, ENT_STRIPE)],
                        tab_sh.at[pl.ds(so, ENT_STRIPE)])

        @pl.when(sid == 1)
        def _():
            pltpu.sync_copy(ent2_hbm, tab_sh.at[pl.ds(ENT_BULK, ENT2_PAD)])

        pltpu.sync_copy(rel_hbm, rel_v)
        plsc.subcore_barrier()

        ins[0].wait()
        gs[0] = pltpu.async_copy(tab_sh.at[idx[0]], val[0], s_g[0])
        r_in.wait()

        def r_slice(k):
            r0 = (k % STEPS) * R_SLICE

            def it(j, _):
                for u in range(R_UNROLL):
                    s = r0 + (j * R_UNROLL + u) * 16
                    v = ridx[pl.ds(s, 16)]
                    rval[pl.ds(s, 16)] = plsc.load_gather(rel_v, [v])
                return 0

            lax.fori_loop(0, R_SLICE // (16 * R_UNROLL), it, 0)

        r_out0 = None
        for k in range(NCHUNK):
            b = k % 2
            nb = (k + 1) % 2
            if k + 1 < NCHUNK:
                ins[k + 1].wait()
                if k >= 1:
                    outs[k - 1].wait()
                gs[k + 1] = pltpu.async_copy(
                    tab_sh.at[idx[nb]], val[nb], s_g[nb])
            if k == STEPS:
                r_in.wait()
                r_out0.wait()
            r_slice(k)
            if k == STEPS - 1:
                r_out0 = pltpu.async_copy(
                    rval, ro_hbm.at[pl.ds(base, HALF_R)], srout)
                r_in = pltpu.async_copy(
                    r_hbm.at[pl.ds(base + HALF_R, HALF_R)], ridx, srin)
            gs[k].wait()
            outs[k] = pltpu.async_copy(val[b], dst_slice(k), s_out[b])
            if k + 2 < NCHUNK:
                ins[k + 2] = pltpu.async_copy(src_slice(k + 2), idx[b], s_in[b])

        r_out1 = pltpu.async_copy(
            rval, ro_hbm.at[pl.ds(base + HALF_R, HALF_R)], srout)
        outs[NCHUNK - 2].wait()
        outs[NCHUNK - 1].wait()
        r_out1.wait()

    return body


_sc_gather = _make_sc_gather()


def kernel(raw_triples, entity_map, relation_map):
    raw_triples = raw_triples.astype(jnp.int32)
    h = raw_triples[:, 0]
    r = raw_triples[:, 1]
    t = raw_triples[:, 2]
    ent = entity_map.astype(jnp.int32)
    ent2 = jnp.pad(ent[ENT_BULK:], (0, ENT2_PAD - (N_ENT - ENT_BULK)))
    rel = jnp.pad(relation_map.astype(jnp.int32), (0, REL_PAD - N_REL))
    h_enc, r_enc, t_enc = _sc_gather(h, r, t, ent, ent2, rel)
    return jnp.stack((h_enc, r_enc, t_enc), axis=1)
